# baseline (speedup 1.0000x reference)
import jax
import jax.numpy as jnp
from jax import lax
from jax.experimental import pallas as pl
from jax.experimental.pallas import tpu as pltpu
from jax.experimental.pallas import tpu_sc as plsc

_CONF_THRES = 0.001
_IOU_THRES = 0.45
_MAX_DET = 100
_MAX_WH = 4096.0
_N = 5000
_NPAD = 5120
_NC = 80
_B = 4
_NEG_INF = float("-inf")
_BIG_I = 2 ** 30

_NW = 8
_PER = 640
_NV = 40
_ROW = 656
_NF = 11


def _prep_body(pred_ref, out_ref):
    p = pred_ref[...]
    cx = p[:, :, 0]
    cy = p[:, :, 1]
    w = p[:, :, 2]
    h = p[:, :, 3]
    obj = p[:, :, 4]
    bx1 = cx - w / 2.0
    by1 = cy - h / 2.0
    bx2 = cx + w / 2.0
    by2 = cy + h / 2.0

    cs = p[:, :, 5:] * obj[:, :, None]
    conf = jnp.max(cs, axis=2)
    cls_iota = lax.broadcasted_iota(jnp.int32, (1, 1, _NC), 2)
    j = jnp.min(jnp.where(cs == conf[:, :, None], cls_iota, _NC), axis=2)
    cls_f = j.astype(jnp.float32)

    off = cls_f * _MAX_WH
    x1 = bx1 + off
    y1 = by1 + off
    x2 = bx2 + off
    y2 = by2 + off
    areas = (x2 - x1) * (y2 - y1)
    s0 = jnp.where(conf > _CONF_THRES, conf, _NEG_INF)

    pad_inf = jnp.full((1, _NPAD - _N), _NEG_INF, jnp.float32)
    pad_z = jnp.zeros((1, _NPAD - _N), jnp.float32)

    def padded(a, pad):
        return jnp.concatenate([a, pad], axis=1)

    out_ref[...] = jnp.stack(
        [padded(s0, pad_inf), padded(x1, pad_z), padded(y1, pad_z),
         padded(x2, pad_z), padded(y2, pad_z), padded(areas, pad_z),
         padded(bx1, pad_z), padded(by1, pad_z), padded(bx2, pad_z),
         padded(by2, pad_z), padded(cls_f, pad_z)], axis=1)


_GDN = lax.GatherDimensionNumbers(offset_dims=(), collapsed_slice_dims=(0,),
                                 start_index_map=(0,))


def _rot(x, sh, iota16):
    perm = jnp.bitwise_and(iota16 + sh, 15)
    return lax.gather(x, perm[:, None], _GDN, (1,),
                      mode=lax.GatherScatterMode.PROMISE_IN_BOUNDS)


def _sc_body(feat_hbm, out_hbm, feats, rows, rec, allrec, shared):
    c = lax.axis_index("c")
    sid = lax.axis_index("s")
    g = sid // _NW
    w = sid - g * _NW
    b = 2 * c + g
    base = w * _PER

    for f in range(_NF):
        pltpu.sync_copy(
            feat_hbm.at[pl.ds((b * _NF + f) * _NPAD + base, _PER)],
            feats.at[pl.ds(f * _ROW, _PER)])

    iota16 = lax.broadcasted_iota(jnp.int32, (16,), 0)
    zeros16 = jnp.zeros((16,), jnp.float32)

    def zero_rows(i, carry):
        rows[pl.ds(i * 16, 16)] = zeros16
        return carry
    lax.fori_loop(0, _MAX_DET, zero_rows, 0)

    def body(k, carry):
        x1w, y1w, x2w, y2w, areaw, miw = carry

        accs = []
        for a in range(4):
            best = jnp.full((16,), _NEG_INF, jnp.float32)
            bestidx = (base + 160 * a) + iota16
            for jv in range(10 * a, 10 * a + 10):
                o = 16 * jv
                sj = feats[pl.ds(o, 16)]
                x1 = feats[pl.ds(1 * _ROW + o, 16)]
                y1 = feats[pl.ds(2 * _ROW + o, 16)]
                x2 = feats[pl.ds(3 * _ROW + o, 16)]
                y2 = feats[pl.ds(4 * _ROW + o, 16)]
                ar = feats[pl.ds(5 * _ROW + o, 16)]
                xx1 = jnp.maximum(x1w, x1)
                yy1 = jnp.maximum(y1w, y1)
                xx2 = jnp.minimum(x2w, x2)
                yy2 = jnp.minimum(y2w, y2)
                inter = jnp.maximum(xx2 - xx1, 0.0) * jnp.maximum(yy2 - yy1, 0.0)
                iou = inter / (areaw + ar - inter + 1e-9)
                idxv = (base + o) + iota16
                sj = jnp.where((iou > _IOU_THRES) | (idxv == miw), _NEG_INF, sj)
                feats[pl.ds(o, 16)] = sj
                upd = sj > best
                best = jnp.where(upd, sj, best)
                bestidx = jnp.where(upd, idxv, bestidx)
            accs.append((best, bestidx))

        def mg_stripe(p, q):
            t = q[0] > p[0]
            return jnp.where(t, q[0], p[0]), jnp.where(t, q[1], p[1])
        m01 = mg_stripe(accs[0], accs[1])
        m23 = mg_stripe(accs[2], accs[3])
        v, ix = mg_stripe(m01, m23)

        for sh in (8, 4, 2, 1):
            v2 = _rot(v, sh, iota16)
            i2 = _rot(ix, sh, iota16)
            take = (v2 > v) | ((v2 == v) & (i2 < ix))
            v = jnp.where(take, v2, v)
            ix = jnp.where(take, i2, ix)
        mv = v[0]
        mi = ix[0]
        li = mi - base

        parts = [jnp.where(iota16 == 0, mv, 0.0)]
        for f in range(1, _NF):
            val = feats[pl.ds(f * _ROW + li, 16)][0]
            parts.append(jnp.where(iota16 == f, val, 0.0))
        parts.append(jnp.where(iota16 == _NF, mi.astype(jnp.float32), 0.0))
        while len(parts) > 1:
            parts = [parts[i] + parts[i + 1] for i in range(0, len(parts) - 1, 2)] \
                + ([parts[-1]] if len(parts) % 2 else [])
        rec[...] = parts[0]

        par = jnp.bitwise_and(k, 1)
        pltpu.sync_copy(rec, shared.at[pl.ds(par * 256 + g * 128 + w * 16, 16)])
        plsc.subcore_barrier()
        pltpu.sync_copy(shared.at[pl.ds(par * 256 + g * 128, 128)], allrec)

        r = [allrec[pl.ds(16 * wi, 16)] for wi in range(_NW)]

        def mg(p, q):
            return jnp.where(q[0] > p[0], q, p)
        gvec = mg(mg(mg(r[0], r[1]), mg(r[2], r[3])),
                  mg(mg(r[4], r[5]), mg(r[6], r[7])))

        gmv = gvec[0]
        x1w_n = gvec[1]
        y1w_n = gvec[2]
        x2w_n = gvec[3]
        y2w_n = gvec[4]
        areaw_n = gvec[5]
        miw_n = gvec[11].astype(jnp.int32)
        keep = gmv > _CONF_THRES

        @pl.when(w == 0)
        def _():
            row = jnp.where(iota16 == 0, gvec[6], zeros16)
            row = jnp.where(iota16 == 1, gvec[7], row)
            row = jnp.where(iota16 == 2, gvec[8], row)
            row = jnp.where(iota16 == 3, gvec[9], row)
            row = jnp.where(iota16 == 4, gmv, row)
            row = jnp.where(iota16 == 5, gvec[10], row)
            row = jnp.where(keep, row, zeros16)
            rows[pl.ds(k * 16, 16)] = row

        return x1w_n, y1w_n, x2w_n, y2w_n, areaw_n, miw_n

    init = (jnp.float32(-1e30), jnp.float32(-1e30), jnp.float32(-1e30),
            jnp.float32(-1e30), jnp.float32(0.0), jnp.int32(-1))
    lax.fori_loop(0, _MAX_DET, body, init)

    @pl.when(w == 0)
    def _():
        pltpu.sync_copy(rows, out_hbm.at[pl.ds(b * _MAX_DET * 16, _MAX_DET * 16)])


def _sc_nms(feat):
    mesh = plsc.VectorSubcoreMesh(core_axis_name="c", subcore_axis_name="s",
                                  num_cores=2, num_subcores=16)
    f = pl.kernel(
        _sc_body,
        out_type=jax.ShapeDtypeStruct((_B * _MAX_DET * 16,), jnp.float32),
        mesh=mesh,
        scratch_types=[
            pltpu.VMEM((_NF * _ROW,), jnp.float32),
            pltpu.VMEM((_MAX_DET * 16,), jnp.float32),
            pltpu.VMEM((16,), jnp.float32),
            pltpu.VMEM((_NW * 16,), jnp.float32),
            pltpu.VMEM_SHARED((2 * 2 * _NW * 16,), jnp.float32),
        ],
    )
    return f(feat)


def kernel(x):
    pred = x[0]
    feat = pl.pallas_call(
        _prep_body,
        grid=(_B,),
        in_specs=[pl.BlockSpec((1, _N, 85), lambda b: (b, 0, 0))],
        out_specs=pl.BlockSpec((1, _NF, _NPAD), lambda b: (b, 0, 0)),
        out_shape=jax.ShapeDtypeStruct((_B, _NF, _NPAD), jnp.float32),
    )(pred)
    feat = feat.reshape(_NF * _B * _NPAD)
    out16 = _sc_nms(feat)
    return out16.reshape(_B, _MAX_DET, 16)[:, :, :6]


# speedup vs baseline: 1.3370x; 1.3370x over previous
import jax
import jax.numpy as jnp
from jax import lax
from jax.experimental import pallas as pl
from jax.experimental.pallas import tpu as pltpu
from jax.experimental.pallas import tpu_sc as plsc

_CONF_THRES = 0.001
_IOU_THRES = 0.45
_MAX_DET = 100
_MAX_WH = 4096.0
_N = 5000
_NPAD = 5120
_NC = 80
_B = 4
_NEG_INF = float("-inf")
_BIG_I = 2 ** 30

_NW = 8
_PER = 640
_NV = 40
_ROW = 656
_NF = 11


def _prep_body(pred_ref, out_ref):
    praw = pred_ref[...]
    pt = jnp.transpose(praw, (0, 2, 1))
    p = jnp.concatenate(
        [pt, jnp.zeros((1, 85, _NPAD - _N), jnp.float32)], axis=2)
    cx = p[:, 0]
    cy = p[:, 1]
    w = p[:, 2]
    h = p[:, 3]
    obj = p[:, 4]
    bx1 = cx - w / 2.0
    by1 = cy - h / 2.0
    bx2 = cx + w / 2.0
    by2 = cy + h / 2.0

    cs = p[:, 5:] * obj[:, None]
    conf = jnp.max(cs, axis=1)
    cls_iota = lax.broadcasted_iota(jnp.int32, (1, _NC, 1), 1)
    j = jnp.min(jnp.where(cs == conf[:, None], cls_iota, _NC), axis=1)
    cls_f = j.astype(jnp.float32)

    off = cls_f * _MAX_WH
    x1 = bx1 + off
    y1 = by1 + off
    x2 = bx2 + off
    y2 = by2 + off
    areas = (x2 - x1) * (y2 - y1)
    s0 = jnp.where(conf > _CONF_THRES, conf, _NEG_INF)

    out_ref[...] = jnp.stack(
        [s0, x1, y1, x2, y2, areas, bx1, by1, bx2, by2, cls_f], axis=1)


_GDN = lax.GatherDimensionNumbers(offset_dims=(), collapsed_slice_dims=(0,),
                                 start_index_map=(0,))


def _rot(x, sh, iota16):
    perm = jnp.bitwise_and(iota16 + sh, 15)
    return lax.gather(x, perm[:, None], _GDN, (1,),
                      mode=lax.GatherScatterMode.PROMISE_IN_BOUNDS)


def _sc_body(feat_hbm, out_hbm, feats, rows, rec, allrec, shared):
    c = lax.axis_index("c")
    sid = lax.axis_index("s")
    g = sid // _NW
    w = sid - g * _NW
    b = 2 * c + g
    base = w * _PER

    for f in range(_NF):
        pltpu.sync_copy(
            feat_hbm.at[pl.ds((b * _NF + f) * _NPAD + base, _PER)],
            feats.at[pl.ds(f * _ROW, _PER)])

    iota16 = lax.broadcasted_iota(jnp.int32, (16,), 0)
    zeros16 = jnp.zeros((16,), jnp.float32)

    def zero_rows(i, carry):
        rows[pl.ds(i * 16, 16)] = zeros16
        return carry
    lax.fori_loop(0, _MAX_DET, zero_rows, 0)

    def body(k, carry):
        x1w, y1w, x2w, y2w, areaw, miw = carry

        accs = []
        for a in range(4):
            best = jnp.full((16,), _NEG_INF, jnp.float32)
            bestidx = (base + 160 * a) + iota16
            for jv in range(10 * a, 10 * a + 10):
                o = 16 * jv
                sj = feats[pl.ds(o, 16)]
                x1 = feats[pl.ds(1 * _ROW + o, 16)]
                y1 = feats[pl.ds(2 * _ROW + o, 16)]
                x2 = feats[pl.ds(3 * _ROW + o, 16)]
                y2 = feats[pl.ds(4 * _ROW + o, 16)]
                ar = feats[pl.ds(5 * _ROW + o, 16)]
                xx1 = jnp.maximum(x1w, x1)
                yy1 = jnp.maximum(y1w, y1)
                xx2 = jnp.minimum(x2w, x2)
                yy2 = jnp.minimum(y2w, y2)
                inter = jnp.maximum(xx2 - xx1, 0.0) * jnp.maximum(yy2 - yy1, 0.0)
                iou = inter / (areaw + ar - inter + 1e-9)
                idxv = (base + o) + iota16
                sj = jnp.where((iou > _IOU_THRES) | (idxv == miw), _NEG_INF, sj)
                feats[pl.ds(o, 16)] = sj
                upd = sj > best
                best = jnp.where(upd, sj, best)
                bestidx = jnp.where(upd, idxv, bestidx)
            accs.append((best, bestidx))

        def mg_stripe(p, q):
            t = q[0] > p[0]
            return jnp.where(t, q[0], p[0]), jnp.where(t, q[1], p[1])
        m01 = mg_stripe(accs[0], accs[1])
        m23 = mg_stripe(accs[2], accs[3])
        v, ix = mg_stripe(m01, m23)

        for sh in (8, 4, 2, 1):
            v2 = _rot(v, sh, iota16)
            i2 = _rot(ix, sh, iota16)
            take = (v2 > v) | ((v2 == v) & (i2 < ix))
            v = jnp.where(take, v2, v)
            ix = jnp.where(take, i2, ix)
        mv = v[0]
        mi = ix[0]
        li = mi - base

        parts = [jnp.where(iota16 == 0, mv, 0.0)]
        for f in range(1, _NF):
            val = feats[pl.ds(f * _ROW + li, 16)][0]
            parts.append(jnp.where(iota16 == f, val, 0.0))
        parts.append(jnp.where(iota16 == _NF, mi.astype(jnp.float32), 0.0))
        while len(parts) > 1:
            parts = [parts[i] + parts[i + 1] for i in range(0, len(parts) - 1, 2)] \
                + ([parts[-1]] if len(parts) % 2 else [])
        rec[...] = parts[0]

        par = jnp.bitwise_and(k, 1)
        pltpu.sync_copy(rec, shared.at[pl.ds(par * 256 + g * 128 + w * 16, 16)])
        plsc.subcore_barrier()
        pltpu.sync_copy(shared.at[pl.ds(par * 256 + g * 128, 128)], allrec)

        r = [allrec[pl.ds(16 * wi, 16)] for wi in range(_NW)]

        def mg(p, q):
            return jnp.where(q[0] > p[0], q, p)
        gvec = mg(mg(mg(r[0], r[1]), mg(r[2], r[3])),
                  mg(mg(r[4], r[5]), mg(r[6], r[7])))

        gmv = gvec[0]
        x1w_n = gvec[1]
        y1w_n = gvec[2]
        x2w_n = gvec[3]
        y2w_n = gvec[4]
        areaw_n = gvec[5]
        miw_n = gvec[11].astype(jnp.int32)
        keep = gmv > _CONF_THRES

        @pl.when(w == 0)
        def _():
            row = jnp.where(iota16 == 0, gvec[6], zeros16)
            row = jnp.where(iota16 == 1, gvec[7], row)
            row = jnp.where(iota16 == 2, gvec[8], row)
            row = jnp.where(iota16 == 3, gvec[9], row)
            row = jnp.where(iota16 == 4, gmv, row)
            row = jnp.where(iota16 == 5, gvec[10], row)
            row = jnp.where(keep, row, zeros16)
            rows[pl.ds(k * 16, 16)] = row

        return x1w_n, y1w_n, x2w_n, y2w_n, areaw_n, miw_n

    init = (jnp.float32(-1e30), jnp.float32(-1e30), jnp.float32(-1e30),
            jnp.float32(-1e30), jnp.float32(0.0), jnp.int32(-1))
    lax.fori_loop(0, _MAX_DET, body, init)

    @pl.when(w == 0)
    def _():
        pltpu.sync_copy(rows, out_hbm.at[pl.ds(b * _MAX_DET * 16, _MAX_DET * 16)])


def _sc_nms(feat):
    mesh = plsc.VectorSubcoreMesh(core_axis_name="c", subcore_axis_name="s",
                                  num_cores=2, num_subcores=16)
    f = pl.kernel(
        _sc_body,
        out_type=jax.ShapeDtypeStruct((_B * _MAX_DET * 16,), jnp.float32),
        mesh=mesh,
        scratch_types=[
            pltpu.VMEM((_NF * _ROW,), jnp.float32),
            pltpu.VMEM((_MAX_DET * 16,), jnp.float32),
            pltpu.VMEM((16,), jnp.float32),
            pltpu.VMEM((_NW * 16,), jnp.float32),
            pltpu.VMEM_SHARED((2 * 2 * _NW * 16,), jnp.float32),
        ],
    )
    return f(feat)


def kernel(x):
    pred = x[0]
    feat = pl.pallas_call(
        _prep_body,
        grid=(_B,),
        in_specs=[pl.BlockSpec((1, _N, 85), lambda b: (b, 0, 0))],
        out_specs=pl.BlockSpec((1, _NF, _NPAD), lambda b: (b, 0, 0)),
        out_shape=jax.ShapeDtypeStruct((_B, _NF, _NPAD), jnp.float32),
    )(pred)
    feat = feat.reshape(_NF * _B * _NPAD)
    out16 = _sc_nms(feat)
    return out16.reshape(_B, _MAX_DET, 16)[:, :, :6]


# speedup vs baseline: 1.6649x; 1.2452x over previous
import jax
import jax.numpy as jnp
from jax import lax
from jax.experimental import pallas as pl
from jax.experimental.pallas import tpu as pltpu
from jax.experimental.pallas import tpu_sc as plsc

_CONF_THRES = 0.001
_IOU_THRES = 0.45
_MAX_DET = 100
_MAX_WH = 4096.0
_N = 5000
_NPAD = 5120
_NC = 80
_B = 4
_NEG_INF = float("-inf")
_BIG_I = 2 ** 30

_NW = 8
_PER = 640
_NV = 40
_ROW = 656
_NF = 11


def _prep_body(pred_ref, out_ref):
    pt = pred_ref[...]
    p = jnp.concatenate(
        [pt, jnp.zeros((1, 85, _NPAD - _N), jnp.float32)], axis=2)
    cx = p[:, 0]
    cy = p[:, 1]
    w = p[:, 2]
    h = p[:, 3]
    obj = p[:, 4]
    bx1 = cx - w / 2.0
    by1 = cy - h / 2.0
    bx2 = cx + w / 2.0
    by2 = cy + h / 2.0

    cs = p[:, 5:] * obj[:, None]
    conf = jnp.max(cs, axis=1)
    cls_iota = lax.broadcasted_iota(jnp.int32, (1, _NC, 1), 1)
    j = jnp.min(jnp.where(cs == conf[:, None], cls_iota, _NC), axis=1)
    cls_f = j.astype(jnp.float32)

    off = cls_f * _MAX_WH
    x1 = bx1 + off
    y1 = by1 + off
    x2 = bx2 + off
    y2 = by2 + off
    areas = (x2 - x1) * (y2 - y1)
    s0 = jnp.where(conf > _CONF_THRES, conf, _NEG_INF)

    out_ref[...] = jnp.stack(
        [s0, x1, y1, x2, y2, areas, bx1, by1, bx2, by2, cls_f], axis=1)


_GDN = lax.GatherDimensionNumbers(offset_dims=(), collapsed_slice_dims=(0,),
                                 start_index_map=(0,))


def _rot(x, sh, iota16):
    perm = jnp.bitwise_and(iota16 + sh, 15)
    return lax.gather(x, perm[:, None], _GDN, (1,),
                      mode=lax.GatherScatterMode.PROMISE_IN_BOUNDS)


def _sc_body(feat_hbm, out_hbm, feats, rows, rec, allrec, shared):
    c = lax.axis_index("c")
    sid = lax.axis_index("s")
    g = sid // _NW
    w = sid - g * _NW
    b = 2 * c + g
    base = w * _PER

    for f in range(_NF):
        pltpu.sync_copy(
            feat_hbm.at[pl.ds((b * _NF + f) * _NPAD + base, _PER)],
            feats.at[pl.ds(f * _ROW, _PER)])

    iota16 = lax.broadcasted_iota(jnp.int32, (16,), 0)
    zeros16 = jnp.zeros((16,), jnp.float32)

    def zero_rows(i, carry):
        rows[pl.ds(i * 16, 16)] = zeros16
        return carry
    lax.fori_loop(0, _MAX_DET, zero_rows, 0)

    def body(k, carry):
        x1w, y1w, x2w, y2w, areaw, miw = carry

        accs = []
        for a in range(4):
            best = jnp.full((16,), _NEG_INF, jnp.float32)
            bestidx = (base + 160 * a) + iota16
            for jv in range(10 * a, 10 * a + 10):
                o = 16 * jv
                sj = feats[pl.ds(o, 16)]
                x1 = feats[pl.ds(1 * _ROW + o, 16)]
                y1 = feats[pl.ds(2 * _ROW + o, 16)]
                x2 = feats[pl.ds(3 * _ROW + o, 16)]
                y2 = feats[pl.ds(4 * _ROW + o, 16)]
                ar = feats[pl.ds(5 * _ROW + o, 16)]
                xx1 = jnp.maximum(x1w, x1)
                yy1 = jnp.maximum(y1w, y1)
                xx2 = jnp.minimum(x2w, x2)
                yy2 = jnp.minimum(y2w, y2)
                inter = jnp.maximum(xx2 - xx1, 0.0) * jnp.maximum(yy2 - yy1, 0.0)
                iou = inter / (areaw + ar - inter + 1e-9)
                idxv = (base + o) + iota16
                sj = jnp.where((iou > _IOU_THRES) | (idxv == miw), _NEG_INF, sj)
                feats[pl.ds(o, 16)] = sj
                upd = sj > best
                best = jnp.where(upd, sj, best)
                bestidx = jnp.where(upd, idxv, bestidx)
            accs.append((best, bestidx))

        def mg_stripe(p, q):
            t = q[0] > p[0]
            return jnp.where(t, q[0], p[0]), jnp.where(t, q[1], p[1])
        m01 = mg_stripe(accs[0], accs[1])
        m23 = mg_stripe(accs[2], accs[3])
        v, ix = mg_stripe(m01, m23)

        for sh in (8, 4, 2, 1):
            v2 = _rot(v, sh, iota16)
            i2 = _rot(ix, sh, iota16)
            take = (v2 > v) | ((v2 == v) & (i2 < ix))
            v = jnp.where(take, v2, v)
            ix = jnp.where(take, i2, ix)
        mv = v[0]
        mi = ix[0]
        li = mi - base

        parts = [jnp.where(iota16 == 0, mv, 0.0)]
        for f in range(1, _NF):
            val = feats[pl.ds(f * _ROW + li, 16)][0]
            parts.append(jnp.where(iota16 == f, val, 0.0))
        parts.append(jnp.where(iota16 == _NF, mi.astype(jnp.float32), 0.0))
        while len(parts) > 1:
            parts = [parts[i] + parts[i + 1] for i in range(0, len(parts) - 1, 2)] \
                + ([parts[-1]] if len(parts) % 2 else [])
        rec[...] = parts[0]

        par = jnp.bitwise_and(k, 1)
        pltpu.sync_copy(rec, shared.at[pl.ds(par * 256 + g * 128 + w * 16, 16)])
        plsc.subcore_barrier()
        pltpu.sync_copy(shared.at[pl.ds(par * 256 + g * 128, 128)], allrec)

        r = [allrec[pl.ds(16 * wi, 16)] for wi in range(_NW)]

        def mg(p, q):
            return jnp.where(q[0] > p[0], q, p)
        gvec = mg(mg(mg(r[0], r[1]), mg(r[2], r[3])),
                  mg(mg(r[4], r[5]), mg(r[6], r[7])))

        gmv = gvec[0]
        x1w_n = gvec[1]
        y1w_n = gvec[2]
        x2w_n = gvec[3]
        y2w_n = gvec[4]
        areaw_n = gvec[5]
        miw_n = gvec[11].astype(jnp.int32)
        keep = gmv > _CONF_THRES

        @pl.when(w == 0)
        def _():
            row = jnp.where(iota16 == 0, gvec[6], zeros16)
            row = jnp.where(iota16 == 1, gvec[7], row)
            row = jnp.where(iota16 == 2, gvec[8], row)
            row = jnp.where(iota16 == 3, gvec[9], row)
            row = jnp.where(iota16 == 4, gmv, row)
            row = jnp.where(iota16 == 5, gvec[10], row)
            row = jnp.where(keep, row, zeros16)
            rows[pl.ds(k * 16, 16)] = row

        return x1w_n, y1w_n, x2w_n, y2w_n, areaw_n, miw_n

    init = (jnp.float32(-1e30), jnp.float32(-1e30), jnp.float32(-1e30),
            jnp.float32(-1e30), jnp.float32(0.0), jnp.int32(-1))
    lax.fori_loop(0, _MAX_DET, body, init)

    @pl.when(w == 0)
    def _():
        pltpu.sync_copy(rows, out_hbm.at[pl.ds(b * _MAX_DET * 16, _MAX_DET * 16)])


def _sc_nms(feat):
    mesh = plsc.VectorSubcoreMesh(core_axis_name="c", subcore_axis_name="s",
                                  num_cores=2, num_subcores=16)
    f = pl.kernel(
        _sc_body,
        out_type=jax.ShapeDtypeStruct((_B * _MAX_DET * 16,), jnp.float32),
        mesh=mesh,
        scratch_types=[
            pltpu.VMEM((_NF * _ROW,), jnp.float32),
            pltpu.VMEM((_MAX_DET * 16,), jnp.float32),
            pltpu.VMEM((16,), jnp.float32),
            pltpu.VMEM((_NW * 16,), jnp.float32),
            pltpu.VMEM_SHARED((2 * 2 * _NW * 16,), jnp.float32),
        ],
    )
    return f(feat)


def kernel(x):
    pred = x[0]
    pt = jnp.transpose(pred, (0, 2, 1))
    feat = pl.pallas_call(
        _prep_body,
        grid=(_B,),
        in_specs=[pl.BlockSpec((1, 85, _N), lambda b: (b, 0, 0))],
        out_specs=pl.BlockSpec((1, _NF, _NPAD), lambda b: (b, 0, 0)),
        out_shape=jax.ShapeDtypeStruct((_B, _NF, _NPAD), jnp.float32),
    )(pt)
    feat = feat.reshape(_NF * _B * _NPAD)
    out16 = _sc_nms(feat)
    return out16.reshape(_B, _MAX_DET, 16)[:, :, :6]


# speedup vs baseline: 1.6662x; 1.0007x over previous
import jax
import jax.numpy as jnp
from jax import lax
from jax.experimental import pallas as pl
from jax.experimental.pallas import tpu as pltpu
from jax.experimental.pallas import tpu_sc as plsc

_CONF_THRES = 0.001
_IOU_THRES = 0.45
_MAX_DET = 100
_MAX_WH = 4096.0
_N = 5000
_NPAD = 5120
_NC = 80
_B = 4
_NEG_INF = float("-inf")

_NW = 8
_PER = 640
_NV = 40
_ROW = 656
_NF = 11


def _prep_body(pred_ref, out_ref):
    pt = pred_ref[...]
    p = jnp.concatenate(
        [pt, jnp.zeros((1, 85, _NPAD - _N), jnp.float32)], axis=2)
    cx = p[:, 0]
    cy = p[:, 1]
    w = p[:, 2]
    h = p[:, 3]
    obj = p[:, 4]
    bx1 = cx - w / 2.0
    by1 = cy - h / 2.0
    bx2 = cx + w / 2.0
    by2 = cy + h / 2.0

    cs = p[:, 5:] * obj[:, None]
    conf = jnp.max(cs, axis=1)
    cls_iota = lax.broadcasted_iota(jnp.int32, (1, _NC, 1), 1)
    j = jnp.min(jnp.where(cs == conf[:, None], cls_iota, _NC), axis=1)
    cls_f = j.astype(jnp.float32)

    off = cls_f * _MAX_WH
    x1 = bx1 + off
    y1 = by1 + off
    x2 = bx2 + off
    y2 = by2 + off
    areas = (x2 - x1) * (y2 - y1)
    s0 = jnp.where(conf > _CONF_THRES, conf, _NEG_INF)

    out_ref[...] = jnp.stack(
        [s0, x1, y1, x2, y2, areas, bx1, by1, bx2, by2, cls_f], axis=1)


_GDN = lax.GatherDimensionNumbers(offset_dims=(), collapsed_slice_dims=(0,),
                                 start_index_map=(0,))


def _rot(x, sh, iota16):
    perm = jnp.bitwise_and(iota16 + sh, 15)
    return lax.gather(x, perm[:, None], _GDN, (1,),
                      mode=lax.GatherScatterMode.PROMISE_IN_BOUNDS)


def _sc_body(feat_hbm, out_hbm, feats, rows, rec, allrec, shared):
    c = lax.axis_index("c")
    sid = lax.axis_index("s")
    g = sid // _NW
    w = sid - g * _NW
    b = 2 * c + g
    base = w * _PER

    for f in range(_NF):
        pltpu.sync_copy(
            feat_hbm.at[pl.ds((b * _NF + f) * _NPAD + base, _PER)],
            feats.at[pl.ds(f * _ROW, _PER)])

    iota16 = lax.broadcasted_iota(jnp.int32, (16,), 0)
    zeros16 = jnp.zeros((16,), jnp.float32)

    def zero_rows(i, carry):
        rows[pl.ds(i * 16, 16)] = zeros16
        return carry
    lax.fori_loop(0, _MAX_DET, zero_rows, 0)

    def body(k, carry):
        x1w, y1w, x2w, y2w, areaw, miw = carry

        accs = []
        for a in range(4):
            best = jnp.full((16,), _NEG_INF, jnp.float32)
            bestidx = (base + 160 * a) + iota16
            for jv in range(10 * a, 10 * a + 10):
                o = 16 * jv
                sj = feats[pl.ds(o, 16)]
                x1 = feats[pl.ds(1 * _ROW + o, 16)]
                y1 = feats[pl.ds(2 * _ROW + o, 16)]
                x2 = feats[pl.ds(3 * _ROW + o, 16)]
                y2 = feats[pl.ds(4 * _ROW + o, 16)]
                ar = feats[pl.ds(5 * _ROW + o, 16)]
                xx1 = jnp.maximum(x1w, x1)
                yy1 = jnp.maximum(y1w, y1)
                xx2 = jnp.minimum(x2w, x2)
                yy2 = jnp.minimum(y2w, y2)
                inter = jnp.maximum(xx2 - xx1, 0.0) * jnp.maximum(yy2 - yy1, 0.0)
                iou = inter / (areaw + ar - inter + 1e-9)
                idxv = (base + o) + iota16
                sj = jnp.where((iou > _IOU_THRES) | (idxv == miw), _NEG_INF, sj)
                feats[pl.ds(o, 16)] = sj
                upd = sj > best
                best = jnp.where(upd, sj, best)
                bestidx = jnp.where(upd, idxv, bestidx)
            accs.append((best, bestidx))

        def mg_stripe(p, q):
            t = q[0] > p[0]
            return jnp.where(t, q[0], p[0]), jnp.where(t, q[1], p[1])
        m01 = mg_stripe(accs[0], accs[1])
        m23 = mg_stripe(accs[2], accs[3])
        v, ix = mg_stripe(m01, m23)

        for sh in (8, 4, 2, 1):
            v2 = _rot(v, sh, iota16)
            i2 = _rot(ix, sh, iota16)
            take = (v2 > v) | ((v2 == v) & (i2 < ix))
            v = jnp.where(take, v2, v)
            ix = jnp.where(take, i2, ix)
        mv = v[0]
        mi = ix[0]
        li = mi - base

        parts = [jnp.where(iota16 == 0, mv, 0.0)]
        for f in range(1, _NF):
            val = feats[pl.ds(f * _ROW + li, 16)][0]
            parts.append(jnp.where(iota16 == f, val, 0.0))
        parts.append(jnp.where(iota16 == _NF, mi.astype(jnp.float32), 0.0))
        while len(parts) > 1:
            parts = [parts[i] + parts[i + 1] for i in range(0, len(parts) - 1, 2)] \
                + ([parts[-1]] if len(parts) % 2 else [])
        rec[...] = parts[0]

        par = jnp.bitwise_and(k, 1)
        pltpu.sync_copy(rec, shared.at[pl.ds(par * 256 + g * 128 + w * 16, 16)])
        plsc.subcore_barrier()
        pltpu.sync_copy(shared.at[pl.ds(par * 256 + g * 128, 128)], allrec)

        r = [allrec[pl.ds(16 * wi, 16)] for wi in range(_NW)]

        def mg(p, q):
            return jnp.where(q[0] > p[0], q, p)
        gvec = mg(mg(mg(r[0], r[1]), mg(r[2], r[3])),
                  mg(mg(r[4], r[5]), mg(r[6], r[7])))

        gmv = gvec[0]
        x1w_n = gvec[1]
        y1w_n = gvec[2]
        x2w_n = gvec[3]
        y2w_n = gvec[4]
        areaw_n = gvec[5]
        miw_n = gvec[11].astype(jnp.int32)
        keep = gmv > _CONF_THRES

        @pl.when(w == 0)
        def _():
            row = jnp.where(iota16 == 0, gvec[6], zeros16)
            row = jnp.where(iota16 == 1, gvec[7], row)
            row = jnp.where(iota16 == 2, gvec[8], row)
            row = jnp.where(iota16 == 3, gvec[9], row)
            row = jnp.where(iota16 == 4, gmv, row)
            row = jnp.where(iota16 == 5, gvec[10], row)
            row = jnp.where(keep, row, zeros16)
            rows[pl.ds(k * 16, 16)] = row

        return x1w_n, y1w_n, x2w_n, y2w_n, areaw_n, miw_n

    init = (jnp.float32(-1e30), jnp.float32(-1e30), jnp.float32(-1e30),
            jnp.float32(-1e30), jnp.float32(0.0), jnp.int32(-1))
    lax.fori_loop(0, _MAX_DET, body, init)

    @pl.when(w == 0)
    def _():
        pltpu.sync_copy(rows, out_hbm.at[pl.ds(b * _MAX_DET * 16, _MAX_DET * 16)])


def _sc_nms(feat):
    mesh = plsc.VectorSubcoreMesh(core_axis_name="c", subcore_axis_name="s",
                                  num_cores=2, num_subcores=16)
    f = pl.kernel(
        _sc_body,
        out_type=jax.ShapeDtypeStruct((_B * _MAX_DET * 16,), jnp.float32),
        mesh=mesh,
        scratch_types=[
            pltpu.VMEM((_NF * _ROW,), jnp.float32),
            pltpu.VMEM((_MAX_DET * 16,), jnp.float32),
            pltpu.VMEM((16,), jnp.float32),
            pltpu.VMEM((_NW * 16,), jnp.float32),
            pltpu.VMEM_SHARED((2 * 2 * _NW * 16,), jnp.float32),
        ],
    )
    return f(feat)


def kernel(x):
    pred = x[0]
    pt = jnp.transpose(pred, (0, 2, 1))
    feat = pl.pallas_call(
        _prep_body,
        grid=(_B,),
        in_specs=[pl.BlockSpec((1, 85, _N), lambda b: (b, 0, 0))],
        out_specs=pl.BlockSpec((1, _NF, _NPAD), lambda b: (b, 0, 0)),
        out_shape=jax.ShapeDtypeStruct((_B, _NF, _NPAD), jnp.float32),
    )(pt)
    feat = feat.reshape(_NF * _B * _NPAD)
    out16 = _sc_nms(feat)
    return out16.reshape(_B, _MAX_DET, 16)[:, :, :6]
